# baseline (speedup 1.0000x reference)
import functools

import jax
import jax.numpy as jnp
from jax import lax
from jax.experimental import pallas as pl
from jax.experimental.pallas import tpu as pltpu
from jax.experimental.pallas import tpu_sc as plsc

N_ATOMS = 3_200_000
N_GRAPHS = 16384
N_ELEM_PAD = 128

NC = 2
NS = 16
NW = NC * NS
CHUNK = N_ATOMS // NW
BLK = 4000
HALO = 16
NBLK = CHUNK // BLK
STEPS = BLK // 16
GROUP = 5
NPAIR = (NBLK - 1) // 2

_mesh = plsc.VectorSubcoreMesh(core_axis_name="c", subcore_axis_name="s")


@functools.partial(
    pl.kernel,
    out_type=jax.ShapeDtypeStruct((NW, N_GRAPHS), jnp.float32),
    mesh=_mesh,
    scratch_types=[
        pltpu.VMEM((N_ELEM_PAD * 16,), jnp.float32),
        pltpu.VMEM((BLK,), jnp.int32),
        pltpu.VMEM((BLK + HALO,), jnp.int32),
        pltpu.VMEM((BLK,), jnp.int32),
        pltpu.VMEM((BLK + HALO,), jnp.int32),
        pltpu.VMEM((N_GRAPHS,), jnp.float32),
        pltpu.VMEM((N_GRAPHS,), jnp.float32),
        pltpu.SemaphoreType.DMA,
        pltpu.SemaphoreType.DMA,
    ],
    compiler_params=pltpu.CompilerParams(needs_layout_passes=False),
)
def _seg_kernel(lin_hbm, an_hbm, bid_hbm, out_hbm,
                lin_v, an0, bid0, an1, bid1, s_acc, b_acc, sem0, sem1):
    wid = lax.axis_index("s") * NC + lax.axis_index("c")
    base = wid * CHUNK

    pltpu.sync_copy(lin_hbm, lin_v)
    lane = lax.iota(jnp.int32, 16)

    zeros16 = jnp.zeros((16,), jnp.float32)

    @plsc.parallel_loop(0, N_GRAPHS // 128, 1, unroll=2)
    def _zero(i):
        o = i * 128
        for k in range(8):
            s_acc[pl.ds(o + k * 16, 16)] = zeros16
            b_acc[pl.ds(o + k * 16, 16)] = zeros16

    def _start(blk_idx, an_b, bid_b, sem):
        start = pl.multiple_of(base + blk_idx * BLK, 8)
        hstart = pl.multiple_of(jnp.maximum(start - HALO, 0), 8)
        pltpu.async_copy(an_hbm.at[pl.ds(start, BLK)], an_b, sem)
        pltpu.async_copy(bid_hbm.at[pl.ds(hstart, HALO)],
                         bid_b.at[pl.ds(0, HALO)], sem)
        pltpu.async_copy(bid_hbm.at[pl.ds(start, BLK)],
                         bid_b.at[pl.ds(HALO, BLK)], sem)

    def _wait(an_b, bid_b, sem):
        pltpu.make_async_copy(an_hbm.at[pl.ds(0, BLK)], an_b, sem).wait()
        pltpu.make_async_copy(bid_hbm.at[pl.ds(0, HALO)],
                              bid_b.at[pl.ds(0, HALO)], sem).wait()
        pltpu.make_async_copy(bid_hbm.at[pl.ds(0, BLK)],
                              bid_b.at[pl.ds(HALO, BLK)], sem).wait()

    def _process(an_b, bid_b, run):
        @plsc.parallel_loop(0, STEPS // GROUP, 1, unroll=3, carry=run)
        def _group(g, run):
            o0 = g * (16 * GROUP)
            bids, prevs, vs, csums = [], [], [], []
            for j in range(GROUP):
                o = o0 + j * 16
                an = an_b[pl.ds(o, 16)]
                bids.append(bid_b[pl.ds(HALO + o, 16)])
                prevs.append(bid_b[pl.ds(HALO + o - 1, 16)])
                v = plsc.load_gather(lin_v, [an * 16 + lane])
                vs.append(v)
                csums.append(plsc.cumsum(v))
            for j in range(GROUP):
                excl = (csums[j] - vs[j]) + run
                is_start = bids[j] != prevs[j]
                plsc.store_scatter(b_acc, [bids[j]], excl, mask=is_start)
                plsc.store_scatter(s_acc, [prevs[j]], excl, mask=is_start)
                run = run + csums[j][15]
            return run

        return _group

    _start(0, an0, bid0, sem0)
    run = jnp.float32(0.0)

    def _pair(p, run):
        _start(2 * p + 1, an1, bid1, sem1)
        _wait(an0, bid0, sem0)

        @pl.when(jnp.logical_and(p == 0, wid == 0))
        def _fix_first_halo():
            first = bid0[pl.ds(HALO, 16)][0]
            bid0[pl.ds(0, HALO)] = jnp.full((HALO,), first, jnp.int32)

        run = _process(an0, bid0, run)
        _start(2 * p + 2, an0, bid0, sem0)
        _wait(an1, bid1, sem1)
        run = _process(an1, bid1, run)
        return run

    run = lax.fori_loop(0, NPAIR, _pair, run)

    _wait(an0, bid0, sem0)
    run = _process(an0, bid0, run)

    last_bid = bid0[pl.ds(HALO + BLK - 16, 16)][15]
    lane0 = lax.iota(jnp.int32, 16) == 0
    plsc.store_scatter(
        s_acc,
        [jnp.full((16,), last_bid, jnp.int32)],
        jnp.full((16,), run, jnp.float32),
        mask=lane0,
    )

    @plsc.parallel_loop(0, N_GRAPHS // 128, 1, unroll=2)
    def _diff(i):
        o = i * 128
        for k in range(8):
            ok = o + k * 16
            b_acc[pl.ds(ok, 16)] = s_acc[pl.ds(ok, 16)] - b_acc[pl.ds(ok, 16)]

    pltpu.sync_copy(b_acc, out_hbm.at[wid])


def _merge(p_ref, o_ref):
    o_ref[...] = jnp.sum(p_ref[...], axis=0, keepdims=True)


@jax.jit
def kernel(lin_ref, atomic_numbers, batch_ids):
    lin_pad = jnp.pad(lin_ref, (0, N_ELEM_PAD - lin_ref.shape[0]))
    lin_pad = jnp.broadcast_to(lin_pad[:, None], (N_ELEM_PAD, 16)).reshape(-1)
    partials = _seg_kernel(lin_pad, atomic_numbers, batch_ids)
    out = pl.pallas_call(
        _merge,
        out_shape=jax.ShapeDtypeStruct((1, N_GRAPHS), jnp.float32),
    )(partials)
    return out.reshape(N_GRAPHS)


# speedup vs baseline: 1.0414x; 1.0414x over previous
import functools

import jax
import jax.numpy as jnp
from jax import lax
from jax.experimental import pallas as pl
from jax.experimental.pallas import tpu as pltpu
from jax.experimental.pallas import tpu_sc as plsc

N_ATOMS = 3_200_000
N_GRAPHS = 16384
N_ELEM_PAD = 128
MAX_ELEM = 118

NC = 2
NS = 16
NW = NC * NS
CHUNK = N_ATOMS // NW
BLK = 4000
HALO = 16
NBLK = CHUNK // BLK
STEPS = BLK // 16
GROUP = 5
NPAIR = (NBLK - 1) // 2

_mesh = plsc.VectorSubcoreMesh(core_axis_name="c", subcore_axis_name="s")


@functools.partial(
    pl.kernel,
    out_type=jax.ShapeDtypeStruct((NW, N_GRAPHS), jnp.float32),
    mesh=_mesh,
    scratch_types=[
        pltpu.VMEM((N_ELEM_PAD * 16,), jnp.float32),
        pltpu.VMEM((N_ELEM_PAD,), jnp.float32),
        pltpu.VMEM((BLK,), jnp.int32),
        pltpu.VMEM((BLK + HALO,), jnp.int32),
        pltpu.VMEM((BLK,), jnp.int32),
        pltpu.VMEM((BLK + HALO,), jnp.int32),
        pltpu.VMEM((N_GRAPHS,), jnp.float32),
        pltpu.VMEM((N_GRAPHS,), jnp.float32),
        pltpu.SemaphoreType.DMA,
        pltpu.SemaphoreType.DMA,
        pltpu.SemaphoreType.DMA,
    ],
    compiler_params=pltpu.CompilerParams(needs_layout_passes=False),
)
def _seg_kernel(lin_hbm, an_hbm, bid_hbm, out_hbm,
                lin_v, lin_s, an0, bid0, an1, bid1, s_acc, b_acc,
                sem0, sem1, seml):
    wid = lax.axis_index("s") * NC + lax.axis_index("c")
    base = wid * CHUNK

    lin_cp = pltpu.async_copy(lin_hbm, lin_s.at[pl.ds(0, MAX_ELEM)], seml)
    lane = lax.iota(jnp.int32, 16)

    zeros16 = jnp.zeros((16,), jnp.float32)

    @plsc.parallel_loop(0, N_GRAPHS // 128, 1, unroll=2)
    def _zero(i):
        o = i * 128
        for k in range(8):
            s_acc[pl.ds(o + k * 16, 16)] = zeros16
            b_acc[pl.ds(o + k * 16, 16)] = zeros16

    def _start(blk_idx, an_b, bid_b, sem):
        start = pl.multiple_of(base + blk_idx * BLK, 8)
        hstart = pl.multiple_of(jnp.maximum(start - HALO, 0), 8)
        pltpu.async_copy(an_hbm.at[pl.ds(start, BLK)], an_b, sem)
        pltpu.async_copy(bid_hbm.at[pl.ds(hstart, HALO)],
                         bid_b.at[pl.ds(0, HALO)], sem)
        pltpu.async_copy(bid_hbm.at[pl.ds(start, BLK)],
                         bid_b.at[pl.ds(HALO, BLK)], sem)

    def _wait(an_b, bid_b, sem):
        pltpu.make_async_copy(an_hbm.at[pl.ds(0, BLK)], an_b, sem).wait()
        pltpu.make_async_copy(bid_hbm.at[pl.ds(0, HALO)],
                              bid_b.at[pl.ds(0, HALO)], sem).wait()
        pltpu.make_async_copy(bid_hbm.at[pl.ds(0, BLK)],
                              bid_b.at[pl.ds(HALO, BLK)], sem).wait()

    def _process(an_b, bid_b, run):
        @plsc.parallel_loop(0, STEPS // GROUP, 1, unroll=2, carry=run)
        def _group(g, run):
            o0 = g * (16 * GROUP)
            bids, prevs, vs, csums = [], [], [], []
            for j in range(GROUP):
                o = o0 + j * 16
                an = an_b[pl.ds(o, 16)]
                bids.append(bid_b[pl.ds(HALO + o, 16)])
                prevs.append(bid_b[pl.ds(HALO + o - 1, 16)])
                v = plsc.load_gather(lin_v, [an * 16 + lane])
                vs.append(v)
                csums.append(plsc.cumsum(v))
            for j in range(GROUP):
                excl = (csums[j] - vs[j]) + run
                is_start = bids[j] != prevs[j]
                plsc.store_scatter(b_acc, [bids[j]], excl, mask=is_start)
                plsc.store_scatter(s_acc, [prevs[j]], excl, mask=is_start)
                run = run + csums[j][15]
            return run

        return _group

    _start(0, an0, bid0, sem0)

    lin_cp.wait()

    @plsc.parallel_loop(0, N_ELEM_PAD // 16, 1)
    def _rep(b):
        for k in range(16):
            lin_v[pl.ds((b * 16 + k) * 16, 16)] = plsc.load_gather(
                lin_s, [jnp.full((16,), b * 16 + k, jnp.int32)])

    run = jnp.float32(0.0)

    def _pair(p, run):
        _start(2 * p + 1, an1, bid1, sem1)
        _wait(an0, bid0, sem0)

        @pl.when(jnp.logical_and(p == 0, wid == 0))
        def _fix_first_halo():
            first = bid0[pl.ds(HALO, 16)][0]
            bid0[pl.ds(0, HALO)] = jnp.full((HALO,), first, jnp.int32)

        run = _process(an0, bid0, run)
        _start(2 * p + 2, an0, bid0, sem0)
        _wait(an1, bid1, sem1)
        run = _process(an1, bid1, run)
        return run

    run = lax.fori_loop(0, NPAIR, _pair, run)

    _wait(an0, bid0, sem0)
    run = _process(an0, bid0, run)

    last_bid = bid0[pl.ds(HALO + BLK - 16, 16)][15]
    lane0 = lax.iota(jnp.int32, 16) == 0
    plsc.store_scatter(
        s_acc,
        [jnp.full((16,), last_bid, jnp.int32)],
        jnp.full((16,), run, jnp.float32),
        mask=lane0,
    )

    @plsc.parallel_loop(0, N_GRAPHS // 128, 1, unroll=2)
    def _diff(i):
        o = i * 128
        for k in range(8):
            ok = o + k * 16
            b_acc[pl.ds(ok, 16)] = s_acc[pl.ds(ok, 16)] - b_acc[pl.ds(ok, 16)]

    pltpu.sync_copy(b_acc, out_hbm.at[wid])


def _merge(p_ref, o_ref):
    o_ref[...] = jnp.sum(p_ref[...], axis=0, keepdims=True)


@jax.jit
def kernel(lin_ref, atomic_numbers, batch_ids):
    partials = _seg_kernel(lin_ref, atomic_numbers, batch_ids)
    out = pl.pallas_call(
        _merge,
        out_shape=jax.ShapeDtypeStruct((1, N_GRAPHS), jnp.float32),
    )(partials)
    return out.reshape(N_GRAPHS)


# speedup vs baseline: 1.0459x; 1.0044x over previous
import functools

import jax
import jax.numpy as jnp
from jax import lax
from jax.experimental import pallas as pl
from jax.experimental.pallas import tpu as pltpu
from jax.experimental.pallas import tpu_sc as plsc

N_ATOMS = 3_200_000
N_GRAPHS = 16384
N_ELEM_PAD = 128
MAX_ELEM = 118

NC = 2
NS = 16
NW = NC * NS
CHUNK = N_ATOMS // NW
BLK = 4000
HALO = 16
NBLK = CHUNK // BLK
STEPS = BLK // 16
GROUP = 5
NPAIR = (NBLK - 1) // 2

_mesh = plsc.VectorSubcoreMesh(core_axis_name="c", subcore_axis_name="s")


@functools.partial(
    pl.kernel,
    out_type=jax.ShapeDtypeStruct((NW, N_GRAPHS), jnp.float32),
    mesh=_mesh,
    scratch_types=[
        pltpu.VMEM((N_ELEM_PAD * 16,), jnp.float32),
        pltpu.VMEM((N_ELEM_PAD,), jnp.float32),
        pltpu.VMEM((BLK,), jnp.int32),
        pltpu.VMEM((BLK + HALO,), jnp.int32),
        pltpu.VMEM((BLK,), jnp.int32),
        pltpu.VMEM((BLK + HALO,), jnp.int32),
        pltpu.VMEM((N_GRAPHS,), jnp.float32),
        pltpu.VMEM((N_GRAPHS,), jnp.float32),
        pltpu.SemaphoreType.DMA,
        pltpu.SemaphoreType.DMA,
        pltpu.SemaphoreType.DMA,
    ],
    compiler_params=pltpu.CompilerParams(needs_layout_passes=False),
)
def _seg_kernel(lin_hbm, an_hbm, bid_hbm, out_hbm,
                lin_v, lin_s, an0, bid0, an1, bid1, s_acc, b_acc,
                sem0, sem1, seml):
    wid = lax.axis_index("s") * NC + lax.axis_index("c")
    base = wid * CHUNK

    lin_cp = pltpu.async_copy(lin_hbm, lin_s.at[pl.ds(0, MAX_ELEM)], seml)
    lane = lax.iota(jnp.int32, 16)
    fifteen = jnp.full((16,), 15, jnp.int32)

    zeros16 = jnp.zeros((16,), jnp.float32)

    @plsc.parallel_loop(0, N_GRAPHS // 128, 1, unroll=2)
    def _zero(i):
        o = i * 128
        for k in range(8):
            s_acc[pl.ds(o + k * 16, 16)] = zeros16
            b_acc[pl.ds(o + k * 16, 16)] = zeros16

    def _start(blk_idx, an_b, bid_b, sem):
        start = pl.multiple_of(base + blk_idx * BLK, 8)
        hstart = pl.multiple_of(jnp.maximum(start - HALO, 0), 8)
        pltpu.async_copy(an_hbm.at[pl.ds(start, BLK)], an_b, sem)
        pltpu.async_copy(bid_hbm.at[pl.ds(hstart, HALO)],
                         bid_b.at[pl.ds(0, HALO)], sem)
        pltpu.async_copy(bid_hbm.at[pl.ds(start, BLK)],
                         bid_b.at[pl.ds(HALO, BLK)], sem)

    def _wait(an_b, bid_b, sem):
        pltpu.make_async_copy(an_hbm.at[pl.ds(0, BLK)], an_b, sem).wait()
        pltpu.make_async_copy(bid_hbm.at[pl.ds(0, HALO)],
                              bid_b.at[pl.ds(0, HALO)], sem).wait()
        pltpu.make_async_copy(bid_hbm.at[pl.ds(0, BLK)],
                              bid_b.at[pl.ds(HALO, BLK)], sem).wait()

    def _process(an_b, bid_b, run):
        @plsc.parallel_loop(0, STEPS // GROUP, 1, unroll=2, carry=run)
        def _group(g, run):
            o0 = g * (16 * GROUP)
            bids, prevs, vs, csums = [], [], [], []
            for j in range(GROUP):
                o = o0 + j * 16
                an = an_b[pl.ds(o, 16)]
                bids.append(bid_b[pl.ds(HALO + o, 16)])
                prevs.append(bid_b[pl.ds(HALO + o - 1, 16)])
                v = plsc.load_gather(lin_v, [an * 16 + lane])
                vs.append(v)
                csums.append(plsc.cumsum(v))
            for j in range(GROUP):
                excl = (csums[j] - vs[j]) + run
                is_start = bids[j] != prevs[j]
                plsc.store_scatter(b_acc, [bids[j]], excl, mask=is_start)
                plsc.store_scatter(s_acc, [prevs[j]], excl, mask=is_start)
                run = run + csums[j].at[fifteen].get(
                    mode="promise_in_bounds")
            return run

        return _group

    _start(0, an0, bid0, sem0)

    lin_cp.wait()

    @plsc.parallel_loop(0, N_ELEM_PAD // 16, 1)
    def _rep(b):
        for k in range(16):
            lin_v[pl.ds((b * 16 + k) * 16, 16)] = plsc.load_gather(
                lin_s, [jnp.full((16,), b * 16 + k, jnp.int32)])

    run = jnp.zeros((16,), jnp.float32)

    def _pair(p, run):
        _start(2 * p + 1, an1, bid1, sem1)
        _wait(an0, bid0, sem0)

        @pl.when(jnp.logical_and(p == 0, wid == 0))
        def _fix_first_halo():
            first = bid0[pl.ds(HALO, 16)][0]
            bid0[pl.ds(0, HALO)] = jnp.full((HALO,), first, jnp.int32)

        run = _process(an0, bid0, run)
        _start(2 * p + 2, an0, bid0, sem0)
        _wait(an1, bid1, sem1)
        run = _process(an1, bid1, run)
        return run

    run = lax.fori_loop(0, NPAIR, _pair, run)

    _wait(an0, bid0, sem0)
    run = _process(an0, bid0, run)

    last_bid = bid0[pl.ds(HALO + BLK - 16, 16)][15]
    lane0 = lax.iota(jnp.int32, 16) == 0
    plsc.store_scatter(
        s_acc,
        [jnp.full((16,), last_bid, jnp.int32)],
        run,
        mask=lane0,
    )

    @plsc.parallel_loop(0, N_GRAPHS // 128, 1, unroll=2)
    def _diff(i):
        o = i * 128
        for k in range(8):
            ok = o + k * 16
            b_acc[pl.ds(ok, 16)] = s_acc[pl.ds(ok, 16)] - b_acc[pl.ds(ok, 16)]

    pltpu.sync_copy(b_acc, out_hbm.at[wid])


def _merge(p_ref, o_ref):
    o_ref[...] = jnp.sum(p_ref[...], axis=0, keepdims=True)


@jax.jit
def kernel(lin_ref, atomic_numbers, batch_ids):
    partials = _seg_kernel(lin_ref, atomic_numbers, batch_ids)
    out = pl.pallas_call(
        _merge,
        out_shape=jax.ShapeDtypeStruct((1, N_GRAPHS), jnp.float32),
    )(partials)
    return out.reshape(N_GRAPHS)
